# baseline (speedup 1.0000x reference)
import functools

import jax
import jax.numpy as jnp
from jax import lax
from jax.experimental import pallas as pl
from jax.experimental.pallas import tpu as pltpu
from jax.experimental.pallas import tpu_sc as plsc

N = 2048
E = 65536
Q = 4096
HID = 256
HEADS = 4
C = 64
BLK = 256
NBLK = N // BLK
F32 = jnp.float32


_NC, _NS = 2, 16
_EC = E // _NS
_GROUP_ROWS = 512
_SPM = _GROUP_ROWS * N
_DUMP = _SPM
_TPW = _SPM // _NS
_ZB = 16384
_CHUNK = 128


def _sc_scatter_body(s_hbm, d_hbm, adj_hbm, adjt_hbm,
                     s_v, d_v, idx_v, ones_v, z_v, shared):
    c = lax.axis_index("c")
    t = lax.axis_index("s")
    pltpu.sync_copy(s_hbm.at[pl.ds(t * _EC, _EC)], s_v)
    pltpu.sync_copy(d_hbm.at[pl.ds(t * _EC, _EC)], d_v)

    @pl.loop(0, _CHUNK, step=16)
    def _(i):
        ones_v[pl.ds(i, 16)] = jnp.full((16,), 1.0, F32)

    @pl.loop(0, _ZB, step=16)
    def _(i):
        z_v[pl.ds(i, 16)] = jnp.zeros((16,), F32)

    for rows_v, cols_v, out_hbm in ((s_v, d_v, adj_hbm), (d_v, s_v, adjt_hbm)):
        for p in range(2):
            base_row = (c * 2 + p) * _GROUP_ROWS

            @pl.loop(0, _TPW, step=_ZB)
            def _(i):
                pltpu.sync_copy(z_v, shared.at[pl.ds(t * _TPW + i, _ZB)])

            plsc.subcore_barrier()

            @pl.loop(0, _EC // _CHUNK)
            def _(j):
                @pl.loop(0, _CHUNK, step=16)
                def _(k):
                    row = rows_v[pl.ds(j * _CHUNK + k, 16)]
                    col = cols_v[pl.ds(j * _CHUNK + k, 16)]
                    loc = (row - base_row) * N + col
                    ok = (row >= base_row) & (row < base_row + _GROUP_ROWS)
                    idx_v[0, pl.ds(k, 16)] = jnp.where(ok, loc, _DUMP)
                pltpu.sync_copy(ones_v, shared.at[idx_v.at[0]], add=True)

            plsc.subcore_barrier()
            pltpu.sync_copy(
                shared.at[pl.ds(t * _TPW, _TPW)],
                out_hbm.at[pl.ds(base_row * N + t * _TPW, _TPW)])
            plsc.subcore_barrier()


def _sc_build_adj(s_arr, d_arr):
    kern = pl.kernel(
        _sc_scatter_body,
        out_type=[jax.ShapeDtypeStruct((N * N,), F32),
                  jax.ShapeDtypeStruct((N * N,), F32)],
        mesh=plsc.VectorSubcoreMesh(core_axis_name="c", subcore_axis_name="s"),
        scratch_types=[
            pltpu.VMEM((_EC,), jnp.int32),
            pltpu.VMEM((_EC,), jnp.int32),
            pltpu.VMEM((1, _CHUNK), jnp.int32),
            pltpu.VMEM((_CHUNK,), F32),
            pltpu.VMEM((_ZB,), F32),
            pltpu.VMEM_SHARED((_SPM + 16,), F32),
        ],
    )
    return kern(s_arr, d_arr)


def _rw_body(adj_ref, out_ref):
    a = adj_ref[...]
    deg = jnp.maximum(jnp.sum(a, axis=1, keepdims=True), 1.0)
    out_ref[...] = a / deg


def _rw_normalize(adj):
    return pl.pallas_call(
        _rw_body,
        grid=(NBLK,),
        in_specs=[pl.BlockSpec((BLK, N), lambda i: (i, 0))],
        out_specs=pl.BlockSpec((BLK, N), lambda i: (i, 0)),
        out_shape=jax.ShapeDtypeStruct((N, N), F32),
    )(adj)


def _mm_body(x_ref, y_ref, out_ref):
    out_ref[...] = jnp.dot(x_ref[...], y_ref[...],
                           preferred_element_type=F32)


def _matmul(x, y):
    return pl.pallas_call(
        _mm_body,
        grid=(NBLK,),
        in_specs=[
            pl.BlockSpec((BLK, N), lambda i: (i, 0)),
            pl.BlockSpec((N, N), lambda i: (0, 0)),
        ],
        out_specs=pl.BlockSpec((BLK, N), lambda i: (i, 0)),
        out_shape=jax.ShapeDtypeStruct((N, N), F32),
    )(x, y)


def _fuse_body(rw_d_ref, rw_r_ref, a2_r_ref, a3_r_ref, a4_r_ref,
               rw_c_ref, a2_c_ref, a3_c_ref, a4_c_ref,
               x_ref, wr_ref, br_ref, wt_ref, wb_ref, bi_ref, out_ref):
    eye = (lax.broadcasted_iota(jnp.int32, (BLK, BLK), 0) ==
           lax.broadcasted_iota(jnp.int32, (BLK, BLK), 1)).astype(F32)

    def diag_mm(xr, yc):
        p = jnp.dot(xr[...], yc[...], preferred_element_type=F32)
        return jnp.sum(p * eye, axis=1)

    d1 = jnp.sum(rw_d_ref[...] * eye, axis=1)
    d2 = diag_mm(rw_r_ref, rw_c_ref)
    d3 = diag_mm(a2_r_ref, rw_c_ref)
    d4 = diag_mm(a2_r_ref, a2_c_ref)
    d5 = diag_mm(a3_r_ref, a2_c_ref)
    d6 = diag_mm(a3_r_ref, a3_c_ref)
    d7 = diag_mm(a4_r_ref, a3_c_ref)
    d8 = diag_mm(a4_r_ref, a4_c_ref)

    wr = wr_ref[...]
    pe = d1[:, None] * wr[0:1, :]
    for k, dk in enumerate((d2, d3, d4, d5, d6, d7, d8)):
        pe = pe + dk[:, None] * wr[k + 1:k + 2, :]
    pe = pe + br_ref[...]
    h0 = (jnp.dot(x_ref[...], wt_ref[...], preferred_element_type=F32)
          + jnp.dot(pe, wb_ref[...], preferred_element_type=F32)
          + bi_ref[...])
    out_ref[...] = h0


def _rwse_h0(rw, a2, a3, a4, x, w_rwse, b_rwse, w_top, w_bot, b_in):
    row = lambda i: (i, 0)
    col = lambda i: (0, i)
    return pl.pallas_call(
        _fuse_body,
        grid=(NBLK,),
        in_specs=[
            pl.BlockSpec((BLK, BLK), lambda i: (i, i)),
            pl.BlockSpec((BLK, N), row),
            pl.BlockSpec((BLK, N), row),
            pl.BlockSpec((BLK, N), row),
            pl.BlockSpec((BLK, N), row),
            pl.BlockSpec((N, BLK), col),
            pl.BlockSpec((N, BLK), col),
            pl.BlockSpec((N, BLK), col),
            pl.BlockSpec((N, BLK), col),
            pl.BlockSpec((BLK, 128), row),
            pl.BlockSpec((8, 16), lambda i: (0, 0)),
            pl.BlockSpec((1, 16), lambda i: (0, 0)),
            pl.BlockSpec((128, HID), lambda i: (0, 0)),
            pl.BlockSpec((16, HID), lambda i: (0, 0)),
            pl.BlockSpec((1, HID), lambda i: (0, 0)),
        ],
        out_specs=pl.BlockSpec((BLK, HID), row),
        out_shape=jax.ShapeDtypeStruct((N, HID), F32),
    )(rw, rw, a2, a3, a4, rw, a2, a3, a4, x, w_rwse, b_rwse, w_top, w_bot, b_in)


def _proj_body(h_ref, wq_ref, bq_ref, wk_ref, bk_ref, wv_ref, bv_ref,
               ws_ref, bs_ref, q_ref, kt_ref, v_ref, hs_ref):
    h = h_ref[...]
    q_ref[...] = jnp.dot(h, wq_ref[...], preferred_element_type=F32) + bq_ref[...]
    kt = lax.dot_general(wk_ref[...], h, (((0,), (1,)), ((), ())),
                         preferred_element_type=F32)
    kt_ref[...] = kt + bk_ref[...].reshape(HID, 1)
    v_ref[...] = jnp.dot(h, wv_ref[...], preferred_element_type=F32) + bv_ref[...]
    hs_ref[...] = jnp.dot(h, ws_ref[...], preferred_element_type=F32) + bs_ref[...]


def _projections(h, wq, bq, wk, bk, wv, bv, ws, bs):
    return pl.pallas_call(
        _proj_body,
        in_specs=[pl.BlockSpec((N, HID), lambda: (0, 0))] +
                 [pl.BlockSpec((HID, HID), lambda: (0, 0)),
                  pl.BlockSpec((1, HID), lambda: (0, 0))] * 4,
        out_specs=[
            pl.BlockSpec((N, HID), lambda: (0, 0)),
            pl.BlockSpec((HID, N), lambda: (0, 0)),
            pl.BlockSpec((N, HID), lambda: (0, 0)),
            pl.BlockSpec((N, HID), lambda: (0, 0)),
        ],
        out_shape=[
            jax.ShapeDtypeStruct((N, HID), F32),
            jax.ShapeDtypeStruct((HID, N), F32),
            jax.ShapeDtypeStruct((N, HID), F32),
            jax.ShapeDtypeStruct((N, HID), F32),
        ],
    )(h, wq, bq, wk, bk, wv, bv, ws, bs)


def _attn_body(q_ref, kt_ref, v_ref, cnt_ref, h_ref, hsk_ref, g_ref, b_ref,
               out_ref, msg_ref):
    cnt = cnt_ref[...]
    has_edge = cnt > 0.0
    scale = 1.0 / jnp.sqrt(jnp.float32(C))
    for hh in range(HEADS):
        qh = q_ref[:, hh * C:(hh + 1) * C]
        kth = kt_ref[hh * C:(hh + 1) * C, :]
        s = jnp.dot(qh, kth, preferred_element_type=F32) * scale
        sm = jnp.where(has_edge, s, -1e30)
        amax = jnp.max(sm, axis=1, keepdims=True)
        amax = jnp.where(amax > -1e29, amax, 0.0)
        e = jnp.exp(jnp.minimum(s - amax, 0.0)) * cnt
        denom = jnp.sum(e, axis=1, keepdims=True)
        vh = v_ref[:, hh * C:(hh + 1) * C]
        o = jnp.dot(e, vh, preferred_element_type=F32)
        msg_ref[:, hh * C:(hh + 1) * C] = o / (denom + 1e-16)
    total = h_ref[...] + hsk_ref[...] + msg_ref[...]
    mu = jnp.mean(total, axis=1, keepdims=True)
    var = jnp.mean((total - mu) ** 2, axis=1, keepdims=True)
    y = (total - mu) / jnp.sqrt(var + 1e-5) * g_ref[...] + b_ref[...]
    out_ref[...] = jnp.maximum(y, 0.0)


def _attention(q, kt, v, adjt, h, hskip, ln_g, ln_b):
    row = lambda i: (i, 0)
    return pl.pallas_call(
        _attn_body,
        grid=(NBLK,),
        in_specs=[
            pl.BlockSpec((BLK, HID), row),
            pl.BlockSpec((HID, N), lambda i: (0, 0)),
            pl.BlockSpec((N, HID), lambda i: (0, 0)),
            pl.BlockSpec((BLK, N), row),
            pl.BlockSpec((BLK, HID), row),
            pl.BlockSpec((BLK, HID), row),
            pl.BlockSpec((1, HID), lambda i: (0, 0)),
            pl.BlockSpec((1, HID), lambda i: (0, 0)),
        ],
        out_specs=pl.BlockSpec((BLK, HID), row),
        out_shape=jax.ShapeDtypeStruct((N, HID), F32),
        scratch_shapes=[pltpu.VMEM((BLK, HID), F32)],
    )(q, kt, v, adjt, h, hskip, ln_g, ln_b)


def _decode_body(hs_ref, hd_ref, out_ref):
    z = jnp.sum(hs_ref[...] * hd_ref[...], axis=1)
    out_ref[...] = 1.0 / (1.0 + jnp.exp(-z))


def _decode(hs, hd):
    return pl.pallas_call(
        _decode_body,
        in_specs=[pl.BlockSpec((Q, HID), lambda: (0, 0)),
                  pl.BlockSpec((Q, HID), lambda: (0, 0))],
        out_specs=pl.BlockSpec((Q,), lambda: (0,)),
        out_shape=jax.ShapeDtypeStruct((Q,), F32),
    )(hs, hd)


def kernel(x, W_rwse, b_rwse, W_in, b_in, layers, edge_index, src, dst):
    s = edge_index[0]
    d = edge_index[1]
    adj_flat, adjt_flat = _sc_build_adj(s, d)
    adj = adj_flat.reshape(N, N)
    adjt = adjt_flat.reshape(N, N)

    rw = _rw_normalize(adj)
    a2 = _matmul(rw, rw)
    a3 = _matmul(a2, rw)
    a4 = _matmul(a2, a2)

    h = _rwse_h0(rw, a2, a3, a4, x,
                 W_rwse, b_rwse.reshape(1, 16),
                 W_in[:128], W_in[128:], b_in.reshape(1, HID))

    for p in layers:
        q, kt, v, hskip = _projections(
            h, p['Wq'], p['bq'].reshape(1, HID), p['Wk'], p['bk'].reshape(1, HID),
            p['Wv'], p['bv'].reshape(1, HID), p['Wskip'], p['bskip'].reshape(1, HID))
        h = _attention(q, kt, v, adjt, h, hskip,
                       p['ln_g'].reshape(1, HID), p['ln_b'].reshape(1, HID))

    hs = jnp.take(h, src, axis=0)
    hd = jnp.take(h, dst, axis=0)
    return _decode(hs, hd)


# speedup vs baseline: 1.0009x; 1.0009x over previous
import functools

import jax
import jax.numpy as jnp
from jax import lax
from jax.experimental import pallas as pl
from jax.experimental.pallas import tpu as pltpu
from jax.experimental.pallas import tpu_sc as plsc

N = 2048
E = 65536
Q = 4096
HID = 256
HEADS = 4
C = 64
BLK = 256
NBLK = N // BLK
F32 = jnp.float32


_NC, _NS = 2, 16
_EC = E // _NS
_GROUP_ROWS = 512
_SPM = _GROUP_ROWS * N
_DUMP = _SPM
_TPW = _SPM // _NS
_ZB = 16384
_CHUNK = 128


def _sc_scatter_body(s_hbm, d_hbm, adj_hbm, adjt_hbm,
                     s_v, d_v, idx_v, ones_v, z_v, shared, zsem, ssem):
    c = lax.axis_index("c")
    t = lax.axis_index("s")
    pltpu.sync_copy(s_hbm.at[pl.ds(t * _EC, _EC)], s_v)
    pltpu.sync_copy(d_hbm.at[pl.ds(t * _EC, _EC)], d_v)

    @pl.loop(0, _CHUNK, step=16)
    def _(i):
        ones_v[pl.ds(i, 16)] = jnp.full((16,), 1.0, F32)

    @pl.loop(0, _ZB, step=16)
    def _(i):
        z_v[pl.ds(i, 16)] = jnp.zeros((16,), F32)

    for rows_v, cols_v, out_hbm in ((s_v, d_v, adj_hbm), (d_v, s_v, adjt_hbm)):
        for p in range(2):
            base_row = (c * 2 + p) * _GROUP_ROWS

            zcopies = [
                pltpu.async_copy(
                    z_v, shared.at[pl.ds(t * _TPW + i * _ZB, _ZB)], zsem)
                for i in range(_TPW // _ZB)
            ]

            @pl.loop(0, _EC // _CHUNK)
            def _(j):
                @pl.loop(0, _CHUNK, step=16)
                def _(k):
                    row = rows_v[pl.ds(j * _CHUNK + k, 16)]
                    col = cols_v[pl.ds(j * _CHUNK + k, 16)]
                    loc = (row - base_row) * N + col
                    ok = (row >= base_row) & (row < base_row + _GROUP_ROWS)
                    idx_v[j, pl.ds(k, 16)] = jnp.where(ok, loc, _DUMP)

            for cp in zcopies:
                cp.wait()
            plsc.subcore_barrier()

            scopies = [
                pltpu.async_copy(ones_v, shared.at[idx_v.at[j]], ssem,
                                 add=True)
                for j in range(_EC // _CHUNK)
            ]
            for cp in scopies:
                cp.wait()
            plsc.subcore_barrier()

            pltpu.sync_copy(
                shared.at[pl.ds(t * _TPW, _TPW)],
                out_hbm.at[pl.ds(base_row * N + t * _TPW, _TPW)])
            plsc.subcore_barrier()


def _sc_build_adj(s_arr, d_arr):
    kern = pl.kernel(
        _sc_scatter_body,
        out_type=[jax.ShapeDtypeStruct((N * N,), F32),
                  jax.ShapeDtypeStruct((N * N,), F32)],
        mesh=plsc.VectorSubcoreMesh(core_axis_name="c", subcore_axis_name="s"),
        scratch_types=[
            pltpu.VMEM((_EC,), jnp.int32),
            pltpu.VMEM((_EC,), jnp.int32),
            pltpu.VMEM((_EC // _CHUNK, _CHUNK), jnp.int32),
            pltpu.VMEM((_CHUNK,), F32),
            pltpu.VMEM((_ZB,), F32),
            pltpu.VMEM_SHARED((_SPM + 16,), F32),
            pltpu.SemaphoreType.DMA,
            pltpu.SemaphoreType.DMA,
        ],
    )
    return kern(s_arr, d_arr)


def _rw_body(adj_ref, out_ref):
    a = adj_ref[...]
    deg = jnp.maximum(jnp.sum(a, axis=1, keepdims=True), 1.0)
    out_ref[...] = a / deg


def _rw_normalize(adj):
    return pl.pallas_call(
        _rw_body,
        grid=(NBLK,),
        in_specs=[pl.BlockSpec((BLK, N), lambda i: (i, 0))],
        out_specs=pl.BlockSpec((BLK, N), lambda i: (i, 0)),
        out_shape=jax.ShapeDtypeStruct((N, N), F32),
    )(adj)


def _mm_body(x_ref, y_ref, out_ref):
    out_ref[...] = jnp.dot(x_ref[...], y_ref[...],
                           preferred_element_type=F32)


def _matmul(x, y):
    return pl.pallas_call(
        _mm_body,
        grid=(NBLK,),
        in_specs=[
            pl.BlockSpec((BLK, N), lambda i: (i, 0)),
            pl.BlockSpec((N, N), lambda i: (0, 0)),
        ],
        out_specs=pl.BlockSpec((BLK, N), lambda i: (i, 0)),
        out_shape=jax.ShapeDtypeStruct((N, N), F32),
    )(x, y)


def _fuse_body(rw_d_ref, rw_r_ref, a2_r_ref, a3_r_ref, a4_r_ref,
               rw_c_ref, a2_c_ref, a3_c_ref, a4_c_ref,
               x_ref, wr_ref, br_ref, wt_ref, wb_ref, bi_ref, out_ref):
    eye = (lax.broadcasted_iota(jnp.int32, (BLK, BLK), 0) ==
           lax.broadcasted_iota(jnp.int32, (BLK, BLK), 1)).astype(F32)

    def diag_mm(xr, yc):
        p = jnp.dot(xr[...], yc[...], preferred_element_type=F32)
        return jnp.sum(p * eye, axis=1)

    d1 = jnp.sum(rw_d_ref[...] * eye, axis=1)
    d2 = diag_mm(rw_r_ref, rw_c_ref)
    d3 = diag_mm(a2_r_ref, rw_c_ref)
    d4 = diag_mm(a2_r_ref, a2_c_ref)
    d5 = diag_mm(a3_r_ref, a2_c_ref)
    d6 = diag_mm(a3_r_ref, a3_c_ref)
    d7 = diag_mm(a4_r_ref, a3_c_ref)
    d8 = diag_mm(a4_r_ref, a4_c_ref)

    wr = wr_ref[...]
    pe = d1[:, None] * wr[0:1, :]
    for k, dk in enumerate((d2, d3, d4, d5, d6, d7, d8)):
        pe = pe + dk[:, None] * wr[k + 1:k + 2, :]
    pe = pe + br_ref[...]
    h0 = (jnp.dot(x_ref[...], wt_ref[...], preferred_element_type=F32)
          + jnp.dot(pe, wb_ref[...], preferred_element_type=F32)
          + bi_ref[...])
    out_ref[...] = h0


def _rwse_h0(rw, a2, a3, a4, x, w_rwse, b_rwse, w_top, w_bot, b_in):
    row = lambda i: (i, 0)
    col = lambda i: (0, i)
    return pl.pallas_call(
        _fuse_body,
        grid=(NBLK,),
        in_specs=[
            pl.BlockSpec((BLK, BLK), lambda i: (i, i)),
            pl.BlockSpec((BLK, N), row),
            pl.BlockSpec((BLK, N), row),
            pl.BlockSpec((BLK, N), row),
            pl.BlockSpec((BLK, N), row),
            pl.BlockSpec((N, BLK), col),
            pl.BlockSpec((N, BLK), col),
            pl.BlockSpec((N, BLK), col),
            pl.BlockSpec((N, BLK), col),
            pl.BlockSpec((BLK, 128), row),
            pl.BlockSpec((8, 16), lambda i: (0, 0)),
            pl.BlockSpec((1, 16), lambda i: (0, 0)),
            pl.BlockSpec((128, HID), lambda i: (0, 0)),
            pl.BlockSpec((16, HID), lambda i: (0, 0)),
            pl.BlockSpec((1, HID), lambda i: (0, 0)),
        ],
        out_specs=pl.BlockSpec((BLK, HID), row),
        out_shape=jax.ShapeDtypeStruct((N, HID), F32),
    )(rw, rw, a2, a3, a4, rw, a2, a3, a4, x, w_rwse, b_rwse, w_top, w_bot, b_in)


def _proj_body(h_ref, wq_ref, bq_ref, wk_ref, bk_ref, wv_ref, bv_ref,
               ws_ref, bs_ref, q_ref, kt_ref, v_ref, hs_ref):
    h = h_ref[...]
    q_ref[...] = jnp.dot(h, wq_ref[...], preferred_element_type=F32) + bq_ref[...]
    kt = lax.dot_general(wk_ref[...], h, (((0,), (1,)), ((), ())),
                         preferred_element_type=F32)
    kt_ref[...] = kt + bk_ref[...].reshape(HID, 1)
    v_ref[...] = jnp.dot(h, wv_ref[...], preferred_element_type=F32) + bv_ref[...]
    hs_ref[...] = jnp.dot(h, ws_ref[...], preferred_element_type=F32) + bs_ref[...]


def _projections(h, wq, bq, wk, bk, wv, bv, ws, bs):
    return pl.pallas_call(
        _proj_body,
        in_specs=[pl.BlockSpec((N, HID), lambda: (0, 0))] +
                 [pl.BlockSpec((HID, HID), lambda: (0, 0)),
                  pl.BlockSpec((1, HID), lambda: (0, 0))] * 4,
        out_specs=[
            pl.BlockSpec((N, HID), lambda: (0, 0)),
            pl.BlockSpec((HID, N), lambda: (0, 0)),
            pl.BlockSpec((N, HID), lambda: (0, 0)),
            pl.BlockSpec((N, HID), lambda: (0, 0)),
        ],
        out_shape=[
            jax.ShapeDtypeStruct((N, HID), F32),
            jax.ShapeDtypeStruct((HID, N), F32),
            jax.ShapeDtypeStruct((N, HID), F32),
            jax.ShapeDtypeStruct((N, HID), F32),
        ],
    )(h, wq, bq, wk, bk, wv, bv, ws, bs)


def _attn_body(q_ref, kt_ref, v_ref, cnt_ref, h_ref, hsk_ref, g_ref, b_ref,
               out_ref, msg_ref):
    cnt = cnt_ref[...]
    has_edge = cnt > 0.0
    scale = 1.0 / jnp.sqrt(jnp.float32(C))
    for hh in range(HEADS):
        qh = q_ref[:, hh * C:(hh + 1) * C]
        kth = kt_ref[hh * C:(hh + 1) * C, :]
        s = jnp.dot(qh, kth, preferred_element_type=F32) * scale
        sm = jnp.where(has_edge, s, -1e30)
        amax = jnp.max(sm, axis=1, keepdims=True)
        amax = jnp.where(amax > -1e29, amax, 0.0)
        e = jnp.exp(jnp.minimum(s - amax, 0.0)) * cnt
        denom = jnp.sum(e, axis=1, keepdims=True)
        vh = v_ref[:, hh * C:(hh + 1) * C]
        o = jnp.dot(e, vh, preferred_element_type=F32)
        msg_ref[:, hh * C:(hh + 1) * C] = o / (denom + 1e-16)
    total = h_ref[...] + hsk_ref[...] + msg_ref[...]
    mu = jnp.mean(total, axis=1, keepdims=True)
    var = jnp.mean((total - mu) ** 2, axis=1, keepdims=True)
    y = (total - mu) / jnp.sqrt(var + 1e-5) * g_ref[...] + b_ref[...]
    out_ref[...] = jnp.maximum(y, 0.0)


def _attention(q, kt, v, adjt, h, hskip, ln_g, ln_b):
    row = lambda i: (i, 0)
    return pl.pallas_call(
        _attn_body,
        grid=(NBLK,),
        in_specs=[
            pl.BlockSpec((BLK, HID), row),
            pl.BlockSpec((HID, N), lambda i: (0, 0)),
            pl.BlockSpec((N, HID), lambda i: (0, 0)),
            pl.BlockSpec((BLK, N), row),
            pl.BlockSpec((BLK, HID), row),
            pl.BlockSpec((BLK, HID), row),
            pl.BlockSpec((1, HID), lambda i: (0, 0)),
            pl.BlockSpec((1, HID), lambda i: (0, 0)),
        ],
        out_specs=pl.BlockSpec((BLK, HID), row),
        out_shape=jax.ShapeDtypeStruct((N, HID), F32),
        scratch_shapes=[pltpu.VMEM((BLK, HID), F32)],
    )(q, kt, v, adjt, h, hskip, ln_g, ln_b)


def _decode_body(hs_ref, hd_ref, out_ref):
    z = jnp.sum(hs_ref[...] * hd_ref[...], axis=1)
    out_ref[...] = 1.0 / (1.0 + jnp.exp(-z))


def _decode(hs, hd):
    return pl.pallas_call(
        _decode_body,
        in_specs=[pl.BlockSpec((Q, HID), lambda: (0, 0)),
                  pl.BlockSpec((Q, HID), lambda: (0, 0))],
        out_specs=pl.BlockSpec((Q,), lambda: (0,)),
        out_shape=jax.ShapeDtypeStruct((Q,), F32),
    )(hs, hd)


def kernel(x, W_rwse, b_rwse, W_in, b_in, layers, edge_index, src, dst):
    s = edge_index[0]
    d = edge_index[1]
    adj_flat, adjt_flat = _sc_build_adj(s, d)
    adj = adj_flat.reshape(N, N)
    adjt = adjt_flat.reshape(N, N)

    rw = _rw_normalize(adj)
    a2 = _matmul(rw, rw)
    a3 = _matmul(a2, rw)
    a4 = _matmul(a2, a2)

    h = _rwse_h0(rw, a2, a3, a4, x,
                 W_rwse, b_rwse.reshape(1, 16),
                 W_in[:128], W_in[128:], b_in.reshape(1, HID))

    for p in layers:
        q, kt, v, hskip = _projections(
            h, p['Wq'], p['bq'].reshape(1, HID), p['Wk'], p['bk'].reshape(1, HID),
            p['Wv'], p['bv'].reshape(1, HID), p['Wskip'], p['bskip'].reshape(1, HID))
        h = _attention(q, kt, v, adjt, h, hskip,
                       p['ln_g'].reshape(1, HID), p['ln_b'].reshape(1, HID))

    hs = jnp.take(h, src, axis=0)
    hd = jnp.take(h, dst, axis=0)
    return _decode(hs, hd)


# speedup vs baseline: 1.4384x; 1.4372x over previous
import jax
import jax.numpy as jnp
from jax import lax
from jax.experimental import pallas as pl
from jax.experimental.pallas import tpu as pltpu
from jax.experimental.pallas import tpu_sc as plsc

N = 2048
E = 65536
Q = 4096
HID = 256
HEADS = 4
C = 64
BLK = 256
NBLK = N // BLK
F32 = jnp.float32
BF16 = jnp.bfloat16


_NC, _NS = 2, 16
_EC = E // _NS
_GROUP_ROWS = 512
_SPM = _GROUP_ROWS * N
_DUMP = _SPM
_TPW = _SPM // _NS
_ZB = 16384
_CHUNK = 128


def _sc_scatter_body(rows_hbm, cols_hbm, out_hbm,
                     r_v, c_v, idx_v, ones_v, z_v, shared, zsem, ssem):
    c = lax.axis_index("c")
    t = lax.axis_index("s")
    pltpu.sync_copy(rows_hbm.at[pl.ds(t * _EC, _EC)], r_v)
    pltpu.sync_copy(cols_hbm.at[pl.ds(t * _EC, _EC)], c_v)

    @pl.loop(0, _CHUNK, step=16)
    def _(i):
        ones_v[pl.ds(i, 16)] = jnp.full((16,), 1.0, F32)

    @pl.loop(0, _ZB, step=16)
    def _(i):
        z_v[pl.ds(i, 16)] = jnp.zeros((16,), F32)

    for p in range(2):
        base_row = (c * 2 + p) * _GROUP_ROWS

        zcopies = [
            pltpu.async_copy(
                z_v, shared.at[pl.ds(t * _TPW + i * _ZB, _ZB)], zsem)
            for i in range(_TPW // _ZB)
        ]

        @pl.loop(0, _EC // _CHUNK)
        def _(j):
            @pl.loop(0, _CHUNK, step=16)
            def _(k):
                row = r_v[pl.ds(j * _CHUNK + k, 16)]
                col = c_v[pl.ds(j * _CHUNK + k, 16)]
                loc = (row - base_row) * N + col
                ok = (row >= base_row) & (row < base_row + _GROUP_ROWS)
                idx_v[j, pl.ds(k, 16)] = jnp.where(ok, loc, _DUMP)

        for cp in zcopies:
            cp.wait()
        plsc.subcore_barrier()

        scopies = [
            pltpu.async_copy(ones_v, shared.at[idx_v.at[j]], ssem, add=True)
            for j in range(_EC // _CHUNK)
        ]
        for cp in scopies:
            cp.wait()
        plsc.subcore_barrier()

        pltpu.sync_copy(
            shared.at[pl.ds(t * _TPW, _TPW)],
            out_hbm.at[pl.ds(base_row * N + t * _TPW, _TPW)])
        plsc.subcore_barrier()


def _sc_scatter_counts(rows_arr, cols_arr):
    kern = pl.kernel(
        _sc_scatter_body,
        out_type=jax.ShapeDtypeStruct((N * N,), F32),
        mesh=plsc.VectorSubcoreMesh(core_axis_name="c", subcore_axis_name="s"),
        scratch_types=[
            pltpu.VMEM((_EC,), jnp.int32),
            pltpu.VMEM((_EC,), jnp.int32),
            pltpu.VMEM((_EC // _CHUNK, _CHUNK), jnp.int32),
            pltpu.VMEM((_CHUNK,), F32),
            pltpu.VMEM((_ZB,), F32),
            pltpu.VMEM_SHARED((_SPM + 16,), F32),
            pltpu.SemaphoreType.DMA,
            pltpu.SemaphoreType.DMA,
        ],
    )
    return kern(rows_arr, cols_arr)


_GB = (2 * Q) // (_NC * _NS)


def _sc_gather_body(h_hbm, idx_hbm, out_hbm, idx_v, rows_v, sem):
    wid = lax.axis_index("s") * _NC + lax.axis_index("c")
    base = wid * _GB
    pltpu.sync_copy(idx_hbm.at[pl.ds(base, _GB)], idx_v)
    pltpu.async_copy(h_hbm.at[idx_v], rows_v, sem).wait()
    pltpu.sync_copy(rows_v, out_hbm.at[pl.ds(base, _GB)])


def _sc_gather_rows(h, idx):
    kern = pl.kernel(
        _sc_gather_body,
        out_type=jax.ShapeDtypeStruct((2 * Q, HID), F32),
        mesh=plsc.VectorSubcoreMesh(core_axis_name="c", subcore_axis_name="s"),
        scratch_types=[
            pltpu.VMEM((_GB,), jnp.int32),
            pltpu.VMEM((_GB, HID), F32),
            pltpu.SemaphoreType.DMA,
        ],
    )
    return kern(h, idx)


def _rw_body(adj_ref, out_ref):
    a = adj_ref[...]
    deg = jnp.maximum(jnp.sum(a, axis=1, keepdims=True), 1.0)
    out_ref[...] = (a / deg).astype(BF16)


def _rw_normalize(adj):
    return pl.pallas_call(
        _rw_body,
        grid=(NBLK,),
        in_specs=[pl.BlockSpec((BLK, N), lambda i: (i, 0))],
        out_specs=pl.BlockSpec((BLK, N), lambda i: (i, 0)),
        out_shape=jax.ShapeDtypeStruct((N, N), BF16),
    )(adj)


def _mm_body(x_ref, y_ref, out_ref):
    out_ref[...] = jnp.dot(x_ref[...], y_ref[...],
                           preferred_element_type=F32).astype(BF16)


def _matmul(x, y):
    return pl.pallas_call(
        _mm_body,
        grid=(NBLK,),
        in_specs=[
            pl.BlockSpec((BLK, N), lambda i: (i, 0)),
            pl.BlockSpec((N, N), lambda i: (0, 0)),
        ],
        out_specs=pl.BlockSpec((BLK, N), lambda i: (i, 0)),
        out_shape=jax.ShapeDtypeStruct((N, N), BF16),
    )(x, y)


def _fuse_body(rw_d_ref, rw_r_ref, a2_r_ref, a3_r_ref, a4_r_ref,
               rw_c_ref, a2_c_ref, a3_c_ref, a4_c_ref,
               x_ref, wr_ref, br_ref, wt_ref, wb_ref, bi_ref, out_ref):
    eye = (lax.broadcasted_iota(jnp.int32, (BLK, BLK), 0) ==
           lax.broadcasted_iota(jnp.int32, (BLK, BLK), 1)).astype(F32)

    def diag_mm(xr, yc):
        p = jnp.dot(xr[...], yc[...], preferred_element_type=F32)
        return jnp.sum(p * eye, axis=1)

    d1 = jnp.sum(rw_d_ref[...].astype(F32) * eye, axis=1)
    d2 = diag_mm(rw_r_ref, rw_c_ref)
    d3 = diag_mm(a2_r_ref, rw_c_ref)
    d4 = diag_mm(a2_r_ref, a2_c_ref)
    d5 = diag_mm(a3_r_ref, a2_c_ref)
    d6 = diag_mm(a3_r_ref, a3_c_ref)
    d7 = diag_mm(a4_r_ref, a3_c_ref)
    d8 = diag_mm(a4_r_ref, a4_c_ref)

    wr = wr_ref[...]
    pe = d1[:, None] * wr[0:1, :]
    for k, dk in enumerate((d2, d3, d4, d5, d6, d7, d8)):
        pe = pe + dk[:, None] * wr[k + 1:k + 2, :]
    pe = pe + br_ref[...]
    h0 = (jnp.dot(x_ref[...].astype(BF16), wt_ref[...].astype(BF16),
                  preferred_element_type=F32)
          + jnp.dot(pe.astype(BF16), wb_ref[...].astype(BF16),
                    preferred_element_type=F32)
          + bi_ref[...])
    out_ref[...] = h0


def _rwse_h0(rw, a2, a3, a4, x, w_rwse, b_rwse, w_top, w_bot, b_in):
    row = lambda i: (i, 0)
    col = lambda i: (0, i)
    return pl.pallas_call(
        _fuse_body,
        grid=(NBLK,),
        in_specs=[
            pl.BlockSpec((BLK, BLK), lambda i: (i, i)),
            pl.BlockSpec((BLK, N), row),
            pl.BlockSpec((BLK, N), row),
            pl.BlockSpec((BLK, N), row),
            pl.BlockSpec((BLK, N), row),
            pl.BlockSpec((N, BLK), col),
            pl.BlockSpec((N, BLK), col),
            pl.BlockSpec((N, BLK), col),
            pl.BlockSpec((N, BLK), col),
            pl.BlockSpec((BLK, 128), row),
            pl.BlockSpec((8, 16), lambda i: (0, 0)),
            pl.BlockSpec((1, 16), lambda i: (0, 0)),
            pl.BlockSpec((128, HID), lambda i: (0, 0)),
            pl.BlockSpec((16, HID), lambda i: (0, 0)),
            pl.BlockSpec((1, HID), lambda i: (0, 0)),
        ],
        out_specs=pl.BlockSpec((BLK, HID), row),
        out_shape=jax.ShapeDtypeStruct((N, HID), F32),
    )(rw, rw, a2, a3, a4, rw, a2, a3, a4, x, w_rwse, b_rwse, w_top, w_bot,
      b_in)


def _proj_body(h_ref, wq_ref, bq_ref, wk_ref, bk_ref, wv_ref, bv_ref,
               ws_ref, bs_ref, q_ref, kt_ref, v_ref, hs_ref):
    h = h_ref[...].astype(BF16)
    q = jnp.dot(h, wq_ref[...].astype(BF16), preferred_element_type=F32)
    q_ref[...] = (q + bq_ref[...]).astype(BF16)
    kt = lax.dot_general(wk_ref[...].astype(BF16), h, (((0,), (1,)), ((), ())),
                         preferred_element_type=F32)
    kt_ref[...] = (kt + bk_ref[...].reshape(HID, 1)).astype(BF16)
    v = jnp.dot(h, wv_ref[...].astype(BF16), preferred_element_type=F32)
    v_ref[...] = (v + bv_ref[...]).astype(BF16)
    hs_ref[...] = (jnp.dot(h, ws_ref[...].astype(BF16),
                           preferred_element_type=F32) + bs_ref[...])


def _projections(h, wq, bq, wk, bk, wv, bv, ws, bs):
    return pl.pallas_call(
        _proj_body,
        in_specs=[pl.BlockSpec((N, HID), lambda: (0, 0))] +
                 [pl.BlockSpec((HID, HID), lambda: (0, 0)),
                  pl.BlockSpec((1, HID), lambda: (0, 0))] * 4,
        out_specs=[
            pl.BlockSpec((N, HID), lambda: (0, 0)),
            pl.BlockSpec((HID, N), lambda: (0, 0)),
            pl.BlockSpec((N, HID), lambda: (0, 0)),
            pl.BlockSpec((N, HID), lambda: (0, 0)),
        ],
        out_shape=[
            jax.ShapeDtypeStruct((N, HID), BF16),
            jax.ShapeDtypeStruct((HID, N), BF16),
            jax.ShapeDtypeStruct((N, HID), BF16),
            jax.ShapeDtypeStruct((N, HID), F32),
        ],
    )(h, wq, bq, wk, bk, wv, bv, ws, bs)


def _attn_body(q_ref, kt_ref, v_ref, cnt_ref, h_ref, hsk_ref, g_ref, b_ref,
               out_ref, msg_ref):
    cnt = cnt_ref[...]
    has_edge = cnt > 0.0
    scale = 1.0 / jnp.sqrt(jnp.float32(C))
    for hh in range(HEADS):
        qh = q_ref[:, hh * C:(hh + 1) * C]
        kth = kt_ref[hh * C:(hh + 1) * C, :]
        s = jnp.dot(qh, kth, preferred_element_type=F32) * scale
        sm = jnp.where(has_edge, s, -1e30)
        amax = jnp.max(sm, axis=1, keepdims=True)
        amax = jnp.where(amax > -1e29, amax, 0.0)
        e = jnp.exp(jnp.minimum(s - amax, 0.0)) * cnt
        denom = jnp.sum(e, axis=1, keepdims=True)
        vh = v_ref[:, hh * C:(hh + 1) * C]
        o = jnp.dot(e.astype(BF16), vh, preferred_element_type=F32)
        msg_ref[:, hh * C:(hh + 1) * C] = o / (denom + 1e-16)
    total = h_ref[...] + hsk_ref[...] + msg_ref[...]
    mu = jnp.mean(total, axis=1, keepdims=True)
    var = jnp.mean((total - mu) ** 2, axis=1, keepdims=True)
    y = (total - mu) / jnp.sqrt(var + 1e-5) * g_ref[...] + b_ref[...]
    out_ref[...] = jnp.maximum(y, 0.0)


def _attention(q, kt, v, adjt, h, hskip, ln_g, ln_b):
    row = lambda i: (i, 0)
    return pl.pallas_call(
        _attn_body,
        grid=(NBLK,),
        in_specs=[
            pl.BlockSpec((BLK, HID), row),
            pl.BlockSpec((HID, N), lambda i: (0, 0)),
            pl.BlockSpec((N, HID), lambda i: (0, 0)),
            pl.BlockSpec((BLK, N), row),
            pl.BlockSpec((BLK, HID), row),
            pl.BlockSpec((BLK, HID), row),
            pl.BlockSpec((1, HID), lambda i: (0, 0)),
            pl.BlockSpec((1, HID), lambda i: (0, 0)),
        ],
        out_specs=pl.BlockSpec((BLK, HID), row),
        out_shape=jax.ShapeDtypeStruct((N, HID), F32),
        scratch_shapes=[pltpu.VMEM((BLK, HID), F32)],
    )(q, kt, v, adjt, h, hskip, ln_g, ln_b)


def _decode_body(hs_ref, hd_ref, out_ref):
    z = jnp.sum(hs_ref[...] * hd_ref[...], axis=1)
    out_ref[...] = 1.0 / (1.0 + jnp.exp(-z))


def _decode(pairs):
    return pl.pallas_call(
        _decode_body,
        grid=(1,),
        in_specs=[pl.BlockSpec((Q, HID), lambda i: (0, 0)),
                  pl.BlockSpec((Q, HID), lambda i: (1, 0))],
        out_specs=pl.BlockSpec((Q,), lambda i: (0,)),
        out_shape=jax.ShapeDtypeStruct((Q,), F32),
    )(pairs, pairs)


def kernel(x, W_rwse, b_rwse, W_in, b_in, layers, edge_index, src, dst):
    s = edge_index[0]
    d = edge_index[1]
    adj = _sc_scatter_counts(s, d).reshape(N, N)
    adjt = _sc_scatter_counts(d, s).reshape(N, N)

    rw = _rw_normalize(adj)
    a2 = _matmul(rw, rw)
    a3 = _matmul(a2, rw)
    a4 = _matmul(a2, a2)

    h = _rwse_h0(rw, a2, a3, a4, x,
                 W_rwse, b_rwse.reshape(1, 16),
                 W_in[:128], W_in[128:], b_in.reshape(1, HID))

    for p in layers:
        q, kt, v, hskip = _projections(
            h, p['Wq'], p['bq'].reshape(1, HID), p['Wk'], p['bk'].reshape(1, HID),
            p['Wv'], p['bv'].reshape(1, HID), p['Wskip'], p['bskip'].reshape(1, HID))
        h = _attention(q, kt, v, adjt, h, hskip,
                       p['ln_g'].reshape(1, HID), p['ln_b'].reshape(1, HID))

    pairs = _sc_gather_rows(h, jnp.concatenate([src, dst]))
    return _decode(pairs)


# speedup vs baseline: 1.4384x; 1.0000x over previous
import jax
import jax.numpy as jnp
from jax import lax
from jax.experimental import pallas as pl
from jax.experimental.pallas import tpu as pltpu
from jax.experimental.pallas import tpu_sc as plsc

N = 2048
E = 65536
Q = 4096
HID = 256
HEADS = 4
C = 64
BLK = 256
NBLK = N // BLK
F32 = jnp.float32
BF16 = jnp.bfloat16


_NC, _NS = 2, 16
_EC = E // _NS
_GROUP_ROWS = 512
_SPM = _GROUP_ROWS * N
_TPW = _SPM // _NS
_ZB = 16384
_CHUNK = 128


def _sc_scatter_body(rows_hbm, cols_hbm, out_hbm,
                     r_v, c_v, idx_v, val_v, z_v, shared, zsem, ssem):
    c = lax.axis_index("c")
    t = lax.axis_index("s")
    pltpu.sync_copy(rows_hbm.at[pl.ds(t * _EC, _EC)], r_v)
    pltpu.sync_copy(cols_hbm.at[pl.ds(t * _EC, _EC)], c_v)

    @pl.loop(0, _ZB, step=16)
    def _(i):
        z_v[pl.ds(i, 16)] = jnp.zeros((16,), F32)

    for p in range(2):
        base_row = (c * 2 + p) * _GROUP_ROWS

        zcopies = [
            pltpu.async_copy(
                z_v, shared.at[pl.ds(t * _TPW + i * _ZB, _ZB)], zsem)
            for i in range(_TPW // _ZB)
        ]

        @pl.loop(0, _EC // _CHUNK)
        def _(j):
            @pl.loop(0, _CHUNK, step=16)
            def _(k):
                row = r_v[pl.ds(j * _CHUNK + k, 16)]
                col = c_v[pl.ds(j * _CHUNK + k, 16)]
                loc = (row - base_row) * N + col
                ok = (row >= base_row) & (row < base_row + _GROUP_ROWS)
                idx_v[j, pl.ds(k, 16)] = jnp.where(ok, loc, 0)
                val_v[j, pl.ds(k, 16)] = jnp.where(ok, 1.0, 0.0)

        for cp in zcopies:
            cp.wait()
        plsc.subcore_barrier()

        scopies = [
            pltpu.async_copy(val_v.at[j], shared.at[idx_v.at[j]], ssem,
                             add=True)
            for j in range(_EC // _CHUNK)
        ]
        for cp in scopies:
            cp.wait()
        plsc.subcore_barrier()

        pltpu.sync_copy(
            shared.at[pl.ds(t * _TPW, _TPW)],
            out_hbm.at[pl.ds(base_row * N + t * _TPW, _TPW)])
        plsc.subcore_barrier()


def _sc_scatter_counts(rows_arr, cols_arr):
    kern = pl.kernel(
        _sc_scatter_body,
        out_type=jax.ShapeDtypeStruct((N * N,), F32),
        mesh=plsc.VectorSubcoreMesh(core_axis_name="c", subcore_axis_name="s"),
        scratch_types=[
            pltpu.VMEM((_EC,), jnp.int32),
            pltpu.VMEM((_EC,), jnp.int32),
            pltpu.VMEM((_EC // _CHUNK, _CHUNK), jnp.int32),
            pltpu.VMEM((_EC // _CHUNK, _CHUNK), F32),
            pltpu.VMEM((_ZB,), F32),
            pltpu.VMEM_SHARED((_SPM,), F32),
            pltpu.SemaphoreType.DMA,
            pltpu.SemaphoreType.DMA,
        ],
    )
    return kern(rows_arr, cols_arr)


_GB = (2 * Q) // (_NC * _NS)


def _sc_gather_body(h_hbm, idx_hbm, out_hbm, idx_v, rows_v, sem):
    wid = lax.axis_index("s") * _NC + lax.axis_index("c")
    base = wid * _GB
    pltpu.sync_copy(idx_hbm.at[pl.ds(base, _GB)], idx_v)
    pltpu.async_copy(h_hbm.at[idx_v], rows_v, sem).wait()
    pltpu.sync_copy(rows_v, out_hbm.at[pl.ds(base, _GB)])


def _sc_gather_rows(h, idx):
    kern = pl.kernel(
        _sc_gather_body,
        out_type=jax.ShapeDtypeStruct((2 * Q, HID), F32),
        mesh=plsc.VectorSubcoreMesh(core_axis_name="c", subcore_axis_name="s"),
        scratch_types=[
            pltpu.VMEM((_GB,), jnp.int32),
            pltpu.VMEM((_GB, HID), F32),
            pltpu.SemaphoreType.DMA,
        ],
    )
    return kern(h, idx)


def _rw_body(adj_ref, out_ref):
    a = adj_ref[...]
    deg = jnp.maximum(jnp.sum(a, axis=1, keepdims=True), 1.0)
    out_ref[...] = (a / deg).astype(BF16)


def _rw_normalize(adj):
    return pl.pallas_call(
        _rw_body,
        grid=(NBLK,),
        in_specs=[pl.BlockSpec((BLK, N), lambda i: (i, 0))],
        out_specs=pl.BlockSpec((BLK, N), lambda i: (i, 0)),
        out_shape=jax.ShapeDtypeStruct((N, N), BF16),
    )(adj)


def _mm_body(x_ref, y_ref, out_ref):
    out_ref[...] = jnp.dot(x_ref[...], y_ref[...],
                           preferred_element_type=F32).astype(BF16)


def _matmul(x, y):
    return pl.pallas_call(
        _mm_body,
        grid=(NBLK,),
        in_specs=[
            pl.BlockSpec((BLK, N), lambda i: (i, 0)),
            pl.BlockSpec((N, N), lambda i: (0, 0)),
        ],
        out_specs=pl.BlockSpec((BLK, N), lambda i: (i, 0)),
        out_shape=jax.ShapeDtypeStruct((N, N), BF16),
    )(x, y)


def _fuse_body(rw_d_ref, rw_r_ref, a2_r_ref, a3_r_ref, a4_r_ref,
               rw_c_ref, a2_c_ref, a3_c_ref, a4_c_ref,
               x_ref, wr_ref, br_ref, wt_ref, wb_ref, bi_ref, out_ref):
    eye = (lax.broadcasted_iota(jnp.int32, (BLK, BLK), 0) ==
           lax.broadcasted_iota(jnp.int32, (BLK, BLK), 1)).astype(F32)

    def diag_mm(xr, yc):
        p = jnp.dot(xr[...], yc[...], preferred_element_type=F32)
        return jnp.sum(p * eye, axis=1)

    d1 = jnp.sum(rw_d_ref[...].astype(F32) * eye, axis=1)
    d2 = diag_mm(rw_r_ref, rw_c_ref)
    d3 = diag_mm(a2_r_ref, rw_c_ref)
    d4 = diag_mm(a2_r_ref, a2_c_ref)
    d5 = diag_mm(a3_r_ref, a2_c_ref)
    d6 = diag_mm(a3_r_ref, a3_c_ref)
    d7 = diag_mm(a4_r_ref, a3_c_ref)
    d8 = diag_mm(a4_r_ref, a4_c_ref)

    wr = wr_ref[...]
    pe = d1[:, None] * wr[0:1, :]
    for k, dk in enumerate((d2, d3, d4, d5, d6, d7, d8)):
        pe = pe + dk[:, None] * wr[k + 1:k + 2, :]
    pe = pe + br_ref[...]
    h0 = (jnp.dot(x_ref[...].astype(BF16), wt_ref[...].astype(BF16),
                  preferred_element_type=F32)
          + jnp.dot(pe.astype(BF16), wb_ref[...].astype(BF16),
                    preferred_element_type=F32)
          + bi_ref[...])
    out_ref[...] = h0


def _rwse_h0(rw, a2, a3, a4, x, w_rwse, b_rwse, w_top, w_bot, b_in):
    row = lambda i: (i, 0)
    col = lambda i: (0, i)
    return pl.pallas_call(
        _fuse_body,
        grid=(NBLK,),
        in_specs=[
            pl.BlockSpec((BLK, BLK), lambda i: (i, i)),
            pl.BlockSpec((BLK, N), row),
            pl.BlockSpec((BLK, N), row),
            pl.BlockSpec((BLK, N), row),
            pl.BlockSpec((BLK, N), row),
            pl.BlockSpec((N, BLK), col),
            pl.BlockSpec((N, BLK), col),
            pl.BlockSpec((N, BLK), col),
            pl.BlockSpec((N, BLK), col),
            pl.BlockSpec((BLK, 128), row),
            pl.BlockSpec((8, 16), lambda i: (0, 0)),
            pl.BlockSpec((1, 16), lambda i: (0, 0)),
            pl.BlockSpec((128, HID), lambda i: (0, 0)),
            pl.BlockSpec((16, HID), lambda i: (0, 0)),
            pl.BlockSpec((1, HID), lambda i: (0, 0)),
        ],
        out_specs=pl.BlockSpec((BLK, HID), row),
        out_shape=jax.ShapeDtypeStruct((N, HID), F32),
    )(rw, rw, a2, a3, a4, rw, a2, a3, a4, x, w_rwse, b_rwse, w_top, w_bot,
      b_in)


def _proj_body(h_ref, wq_ref, bq_ref, wk_ref, bk_ref, wv_ref, bv_ref,
               ws_ref, bs_ref, q_ref, kt_ref, v_ref, hs_ref):
    h = h_ref[...].astype(BF16)
    q = jnp.dot(h, wq_ref[...].astype(BF16), preferred_element_type=F32)
    q_ref[...] = (q + bq_ref[...]).astype(BF16)
    kt = lax.dot_general(wk_ref[...].astype(BF16), h, (((0,), (1,)), ((), ())),
                         preferred_element_type=F32)
    kt_ref[...] = (kt + bk_ref[...].reshape(HID, 1)).astype(BF16)
    v = jnp.dot(h, wv_ref[...].astype(BF16), preferred_element_type=F32)
    v_ref[...] = (v + bv_ref[...]).astype(BF16)
    hs_ref[...] = (jnp.dot(h, ws_ref[...].astype(BF16),
                           preferred_element_type=F32) + bs_ref[...])


def _projections(h, wq, bq, wk, bk, wv, bv, ws, bs):
    return pl.pallas_call(
        _proj_body,
        in_specs=[pl.BlockSpec((N, HID), lambda: (0, 0))] +
                 [pl.BlockSpec((HID, HID), lambda: (0, 0)),
                  pl.BlockSpec((1, HID), lambda: (0, 0))] * 4,
        out_specs=[
            pl.BlockSpec((N, HID), lambda: (0, 0)),
            pl.BlockSpec((HID, N), lambda: (0, 0)),
            pl.BlockSpec((N, HID), lambda: (0, 0)),
            pl.BlockSpec((N, HID), lambda: (0, 0)),
        ],
        out_shape=[
            jax.ShapeDtypeStruct((N, HID), BF16),
            jax.ShapeDtypeStruct((HID, N), BF16),
            jax.ShapeDtypeStruct((N, HID), BF16),
            jax.ShapeDtypeStruct((N, HID), F32),
        ],
    )(h, wq, bq, wk, bk, wv, bv, ws, bs)


def _attn_body(q_ref, kt_ref, v_ref, cnt_ref, h_ref, hsk_ref, g_ref, b_ref,
               out_ref, msg_ref):
    cnt = cnt_ref[...]
    has_edge = cnt > 0.0
    scale = 1.0 / jnp.sqrt(jnp.float32(C))
    for hh in range(HEADS):
        qh = q_ref[:, hh * C:(hh + 1) * C]
        kth = kt_ref[hh * C:(hh + 1) * C, :]
        s = jnp.dot(qh, kth, preferred_element_type=F32) * scale
        sm = jnp.where(has_edge, s, -1e30)
        amax = jnp.max(sm, axis=1, keepdims=True)
        amax = jnp.where(amax > -1e29, amax, 0.0)
        e = jnp.exp(jnp.minimum(s - amax, 0.0)) * cnt
        denom = jnp.sum(e, axis=1, keepdims=True)
        vh = v_ref[:, hh * C:(hh + 1) * C]
        o = jnp.dot(e.astype(BF16), vh, preferred_element_type=F32)
        msg_ref[:, hh * C:(hh + 1) * C] = o / (denom + 1e-16)
    total = h_ref[...] + hsk_ref[...] + msg_ref[...]
    mu = jnp.mean(total, axis=1, keepdims=True)
    var = jnp.mean((total - mu) ** 2, axis=1, keepdims=True)
    y = (total - mu) / jnp.sqrt(var + 1e-5) * g_ref[...] + b_ref[...]
    out_ref[...] = jnp.maximum(y, 0.0)


def _attention(q, kt, v, adjt, h, hskip, ln_g, ln_b):
    row = lambda i: (i, 0)
    return pl.pallas_call(
        _attn_body,
        grid=(NBLK,),
        in_specs=[
            pl.BlockSpec((BLK, HID), row),
            pl.BlockSpec((HID, N), lambda i: (0, 0)),
            pl.BlockSpec((N, HID), lambda i: (0, 0)),
            pl.BlockSpec((BLK, N), row),
            pl.BlockSpec((BLK, HID), row),
            pl.BlockSpec((BLK, HID), row),
            pl.BlockSpec((1, HID), lambda i: (0, 0)),
            pl.BlockSpec((1, HID), lambda i: (0, 0)),
        ],
        out_specs=pl.BlockSpec((BLK, HID), row),
        out_shape=jax.ShapeDtypeStruct((N, HID), F32),
        scratch_shapes=[pltpu.VMEM((BLK, HID), F32)],
    )(q, kt, v, adjt, h, hskip, ln_g, ln_b)


def _decode_body(hs_ref, hd_ref, out_ref):
    z = jnp.sum(hs_ref[...] * hd_ref[...], axis=1)
    out_ref[...] = 1.0 / (1.0 + jnp.exp(-z))


def _decode(pairs):
    return pl.pallas_call(
        _decode_body,
        grid=(1,),
        in_specs=[pl.BlockSpec((Q, HID), lambda i: (0, 0)),
                  pl.BlockSpec((Q, HID), lambda i: (1, 0))],
        out_specs=pl.BlockSpec((Q,), lambda i: (0,)),
        out_shape=jax.ShapeDtypeStruct((Q,), F32),
    )(pairs, pairs)


def kernel(x, W_rwse, b_rwse, W_in, b_in, layers, edge_index, src, dst):
    s = edge_index[0]
    d = edge_index[1]
    adj = _sc_scatter_counts(s, d).reshape(N, N)
    adjt = _sc_scatter_counts(d, s).reshape(N, N)

    rw = _rw_normalize(adj)
    a2 = _matmul(rw, rw)
    a3 = _matmul(a2, rw)
    a4 = _matmul(a2, a2)

    h = _rwse_h0(rw, a2, a3, a4, x,
                 W_rwse, b_rwse.reshape(1, 16),
                 W_in[:128], W_in[128:], b_in.reshape(1, HID))

    for p in layers:
        q, kt, v, hskip = _projections(
            h, p['Wq'], p['bq'].reshape(1, HID), p['Wk'], p['bk'].reshape(1, HID),
            p['Wv'], p['bv'].reshape(1, HID), p['Wskip'], p['bskip'].reshape(1, HID))
        h = _attention(q, kt, v, adjt, h, hskip,
                       p['ln_g'].reshape(1, HID), p['ln_b'].reshape(1, HID))

    pairs = _sc_gather_rows(h, jnp.concatenate([src, dst]))
    return _decode(pairs)
